# Initial kernel scaffold; baseline (speedup 1.0000x reference)
import jax
import jax.numpy as jnp
from jax.experimental import pallas as pl


def kernel(x, edge_index, edge_attr, u, batch, W1, b1, g1, be1, W2, b2, g2, be2, W3, b3):
    raise NotImplementedError("write your pallas kernel here")


# speedup vs baseline: 21.1167x; 21.1167x over previous
import functools

import jax
import jax.numpy as jnp
from jax import lax
from jax.experimental import pallas as pl
from jax.experimental.pallas import tpu as pltpu
from jax.experimental.pallas import tpu_sc as plsc

_NC = 2
_NS = 16
_L = 16


def _sc_edge_segment_sums(src, edge_attr, batch, num_segments):
    E, F = edge_attr.shape
    N = batch.shape[0]
    CH = 640
    G = CH // 128
    n_chunks = E // CH
    NW = _NC * _NS
    steps = -(-n_chunks // NW)
    n_seg_grp = num_segments // 128

    zeros_acc = jnp.zeros((num_segments, F), jnp.float32)
    ident = jnp.arange(num_segments, dtype=jnp.int32).reshape(n_seg_grp, 128)

    mesh = plsc.VectorSubcoreMesh(core_axis_name="c", subcore_axis_name="s",
                                  num_cores=_NC, num_subcores=_NS)

    @functools.partial(
        pl.kernel,
        out_type=[
            jax.ShapeDtypeStruct((_NC, num_segments, F), jnp.float32),
            jax.ShapeDtypeStruct((_NC, num_segments, F), jnp.float32),
        ],
        mesh=mesh,
        compiler_params=pltpu.CompilerParams(needs_layout_passes=False,
                                             use_tc_tiling_on_sc=False),
        scratch_types=[
            pltpu.VMEM((N,), jnp.int32),
            pltpu.VMEM((CH,), jnp.int32),
            pltpu.VMEM((G, 128), jnp.int32),
            pltpu.VMEM((CH, F), jnp.float32),
            pltpu.VMEM((num_segments, F), jnp.float32),
            pltpu.VMEM((n_seg_grp, 128), jnp.int32),
            pltpu.VMEM_SHARED((num_segments, F), jnp.float32),
            pltpu.VMEM_SHARED((num_segments, F), jnp.float32),
        ],
    )
    def edge_kernel(batch_hbm, src_hbm, attr_hbm, zeros_hbm, ident_hbm,
                    out_sum, out_cnt,
                    batch_v, src_v, segs_v, attr_v, cnt_v, ident_v,
                    acc_s, acc_c):
        cid = lax.axis_index("c")
        sid = lax.axis_index("s")
        wid = sid * _NC + cid

        @pl.when(sid == 0)
        def _():
            pltpu.sync_copy(zeros_hbm, acc_s)
            pltpu.sync_copy(zeros_hbm, acc_c)

        pltpu.sync_copy(batch_hbm, batch_v)
        pltpu.sync_copy(zeros_hbm, cnt_v)
        pltpu.sync_copy(ident_hbm, ident_v)
        plsc.subcore_barrier()

        lane = lax.iota(jnp.int32, _L)
        ones16 = jnp.ones((_L,), jnp.float32)

        def chunk_body(t, carry):
            k = t * NW + wid

            @pl.when(k < n_chunks)
            def _():
                base = k * CH
                pltpu.sync_copy(src_hbm.at[pl.ds(base, CH)], src_v)
                pltpu.sync_copy(attr_hbm.at[pl.ds(base, CH)], attr_v)
                for g in range(G):
                    for j in range(128 // _L):
                        off = g * 128 + j * _L
                        idx = src_v[pl.ds(off, _L)]
                        segs = plsc.load_gather(batch_v, [idx])
                        segs_v[g, pl.ds(j * _L, _L)] = segs
                        plsc.addupdate_scatter(cnt_v, [segs, lane], ones16)
                for g in range(G):
                    pltpu.sync_copy(attr_v.at[pl.ds(g * 128, 128)],
                                    acc_s.at[segs_v.at[g]], add=True)

            return carry

        lax.fori_loop(0, steps, chunk_body, 0)

        for g in range(n_seg_grp):
            pltpu.sync_copy(cnt_v.at[pl.ds(g * 128, 128)],
                            acc_c.at[ident_v.at[g]], add=True)
        plsc.subcore_barrier()

        @pl.when(sid == 0)
        def _():
            pltpu.sync_copy(acc_s, out_sum.at[cid])
            pltpu.sync_copy(acc_c, out_cnt.at[cid])

    return edge_kernel(batch, src, edge_attr, zeros_acc, ident)


def _tc_node_sums(x, batch3d, num_segments):
    NB, _, CN = batch3d.shape
    D = x.shape[1]

    def body(batch_ref, x_ref, sum_ref, cnt_ref):
        i = pl.program_id(0)

        @pl.when(i == 0)
        def _():
            sum_ref[...] = jnp.zeros_like(sum_ref)
            cnt_ref[...] = jnp.zeros_like(cnt_ref)

        b = batch_ref[0]
        bb = jnp.broadcast_to(b, (num_segments, CN))
        seg_iota = lax.broadcasted_iota(jnp.int32, (num_segments, CN), 0)
        onehot = (bb == seg_iota).astype(jnp.float32)
        sum_ref[...] += jnp.dot(onehot, x_ref[...],
                                preferred_element_type=jnp.float32)
        cnt = jnp.sum(onehot, axis=1, keepdims=True)
        cnt_ref[...] += jnp.broadcast_to(cnt, (num_segments, D))

    return pl.pallas_call(
        body,
        grid=(NB,),
        in_specs=[
            pl.BlockSpec((1, 1, CN), lambda i: (i, 0, 0)),
            pl.BlockSpec((CN, D), lambda i: (i, 0)),
        ],
        out_specs=[
            pl.BlockSpec((num_segments, D), lambda i: (0, 0)),
            pl.BlockSpec((num_segments, D), lambda i: (0, 0)),
        ],
        out_shape=[
            jax.ShapeDtypeStruct((num_segments, D), jnp.float32),
            jax.ShapeDtypeStruct((num_segments, D), jnp.float32),
        ],
    )(batch3d, x)


def _tc_mlp(node_sum, node_cnt, esum, ecnt,
            W1, b1, g1, be1, W2, b2, g2, be2, W3, b3):
    D = node_sum.shape[1]

    def body(ns_ref, nc_ref, es_ref, ec_ref,
             W1_ref, b1_ref, g1_ref, be1_ref,
             W2_ref, b2_ref, g2_ref, be2_ref,
             W3_ref, b3_ref, out_ref):
        ncnt = nc_ref[:, 0:1]
        node_mean = ns_ref[...] / jnp.maximum(ncnt, 1.0)
        es = es_ref[0] + es_ref[1]
        ec = jnp.sum(ec_ref[0] + ec_ref[1], axis=1, keepdims=True)
        edge_mean = es / jnp.maximum(ec, 1.0)

        h = (jnp.dot(node_mean, W1_ref[0:D, :],
                     preferred_element_type=jnp.float32)
             + jnp.dot(edge_mean, W1_ref[D:, :],
                       preferred_element_type=jnp.float32)
             + b1_ref[...])
        m = jnp.mean(h, axis=0, keepdims=True)
        v = jnp.mean((h - m) ** 2, axis=0, keepdims=True)
        h = g1_ref[...] * (h - m) / jnp.sqrt(v + 1e-5) + be1_ref[...]
        h = jnp.maximum(h, 0.0)

        h = jnp.dot(h, W2_ref[...], preferred_element_type=jnp.float32) + b2_ref[...]
        m = jnp.mean(h, axis=0, keepdims=True)
        v = jnp.mean((h - m) ** 2, axis=0, keepdims=True)
        h = g2_ref[...] * (h - m) / jnp.sqrt(v + 1e-5) + be2_ref[...]
        h = jnp.maximum(h, 0.0)

        out_ref[...] = (jnp.dot(h, W3_ref[...],
                                preferred_element_type=jnp.float32)
                        + b3_ref[...])

    B = node_sum.shape[0]
    out_dim = W3.shape[1]
    return pl.pallas_call(
        body,
        out_shape=jax.ShapeDtypeStruct((B, out_dim), jnp.float32),
    )(node_sum, node_cnt, esum, ecnt,
      W1, b1.reshape(1, -1), g1.reshape(1, -1), be1.reshape(1, -1),
      W2, b2.reshape(1, -1), g2.reshape(1, -1), be2.reshape(1, -1),
      W3, b3.reshape(1, -1))


def kernel(x, edge_index, edge_attr, u, batch,
           W1, b1, g1, be1, W2, b2, g2, be2, W3, b3):
    num_segments = u.shape[0]
    src = edge_index[0]
    esum, ecnt = _sc_edge_segment_sums(src, edge_attr, batch, num_segments)

    N = x.shape[0]
    CN = 2000
    batch3d = batch.reshape(N // CN, 1, CN)
    node_sum, node_cnt = _tc_node_sums(x, batch3d, num_segments)

    return _tc_mlp(node_sum, node_cnt, esum, ecnt,
                   W1, b1, g1, be1, W2, b2, g2, be2, W3, b3)
